# Initial kernel scaffold; baseline (speedup 1.0000x reference)
import jax
import jax.numpy as jnp
from jax.experimental import pallas as pl


def kernel(x, topo, edge_index, topo_W1, topo_b1, topo_W2, topo_b2, conv1_W, conv1_b, conv2_W, conv2_b, out_W, out_b):
    raise NotImplementedError("write your pallas kernel here")


# speedup vs baseline: 13.0625x; 13.0625x over previous
import functools

import jax
import jax.numpy as jnp
from jax import lax
from jax.experimental import pallas as pl
from jax.experimental.pallas import tpu as pltpu
from jax.experimental.pallas import tpu_sc as plsc

N_NODES = 10000
N_EDGES = 320000
NODE_IN = 128
TOPO_IN = 16
HIDDEN = 64

NC = 2
NS = 16
NW = NC * NS
EPW = N_EDGES // NW
CH = 80
NCH = EPW // CH
RPT = 640
NP = RPT * NS

_mesh = plsc.VectorSubcoreMesh(
    core_axis_name="c", subcore_axis_name="s", num_cores=NC, num_subcores=NS)
_sc_params = pltpu.CompilerParams(use_tc_tiling_on_sc=False)


@functools.partial(
    pl.kernel,
    out_type=jax.ShapeDtypeStruct((NC, NP, 16), jnp.float32),
    mesh=_mesh,
    scratch_types=[
        pltpu.VMEM_SHARED((NP, 16), jnp.float32),
        pltpu.VMEM((CH,), jnp.int32),
        pltpu.VMEM((CH, 16), jnp.float32),
        pltpu.VMEM((RPT, 16), jnp.float32),
    ],
    compiler_params=_sc_params,
)
def _deg_kernel(dst_hbm, ones_hbm, zeros_hbm, out_hbm, tab, idx_v, ones_v, zb):
    c = lax.axis_index("c")
    s = lax.axis_index("s")
    w = s * NC + c
    pltpu.sync_copy(zeros_hbm, zb)
    pltpu.sync_copy(zb, tab.at[pl.ds(s * RPT, RPT)])
    pltpu.sync_copy(ones_hbm, ones_v)
    plsc.subcore_barrier()

    def body(i, carry):
        base = w * EPW + i * CH
        pltpu.sync_copy(dst_hbm.at[pl.ds(base, CH)], idx_v)
        pltpu.sync_copy(ones_v, tab.at[idx_v], add=True)
        return carry

    lax.fori_loop(0, NCH, body, 0)
    plsc.subcore_barrier()
    pltpu.sync_copy(tab.at[pl.ds(s * RPT, RPT)], zb)
    pltpu.sync_copy(zb, out_hbm.at[c, pl.ds(s * RPT, RPT)])


@functools.partial(
    pl.kernel,
    out_type=jax.ShapeDtypeStruct((NC, NP, HIDDEN), jnp.float32),
    mesh=_mesh,
    scratch_types=[
        pltpu.VMEM_SHARED((NP, HIDDEN), jnp.float32),
        pltpu.VMEM((CH,), jnp.int32),
        pltpu.VMEM((CH,), jnp.int32),
        pltpu.VMEM((CH, HIDDEN), jnp.float32),
        pltpu.VMEM((RPT, HIDDEN), jnp.float32),
        pltpu.SemaphoreType.DMA,
    ],
    compiler_params=_sc_params,
)
def _agg_kernel(g_hbm, src_hbm, dst_hbm, zeros_hbm, out_hbm,
                acc, sidx, didx, rows, zb, sem):
    c = lax.axis_index("c")
    s = lax.axis_index("s")
    w = s * NC + c
    pltpu.sync_copy(zeros_hbm, zb)
    pltpu.sync_copy(zb, acc.at[pl.ds(s * RPT, RPT)])
    plsc.subcore_barrier()

    def body(i, carry):
        base = w * EPW + i * CH
        pltpu.sync_copy(src_hbm.at[pl.ds(base, CH)], sidx)
        pltpu.sync_copy(dst_hbm.at[pl.ds(base, CH)], didx)
        pltpu.async_copy(g_hbm.at[sidx], rows, sem).wait()
        pltpu.sync_copy(rows, acc.at[didx], add=True)
        return carry

    lax.fori_loop(0, NCH, body, 0)
    plsc.subcore_barrier()
    pltpu.sync_copy(acc.at[pl.ds(s * RPT, RPT)], zb)
    pltpu.sync_copy(zb, out_hbm.at[c, pl.ds(s * RPT, RPT)])


BN = 1000
_GRID = N_NODES // BN


def _stage1_body(x_ref, topo_ref, degp_ref, tw1, tb1, tw2, tb2, c1w,
                 g1_ref, dinv_ref):
    z = jnp.maximum(topo_ref[...] @ tw1[...] + tb1[...], 0.0)
    z = jnp.maximum(z @ tw2[...] + tb2[...], 0.0)
    deg = degp_ref[0, :, 0:1] + degp_ref[1, :, 0:1] + 1.0
    dinv = lax.rsqrt(deg)
    h = jnp.concatenate([x_ref[...], z], axis=1)
    g1_ref[...] = dinv * (h @ c1w[...])
    dinv_ref[...] = dinv


def _stage2_body(p_ref, g1_ref, dinv_ref, c2w, c1b, g2_ref):
    dinv = dinv_ref[...]
    h1 = jnp.maximum(dinv * (p_ref[0] + p_ref[1] + g1_ref[...]) + c1b[...], 0.0)
    g2_ref[...] = dinv * (h1 @ c2w[...])


def _stage3_body(p_ref, g2_ref, dinv_ref, c2b, ow, ob, y_ref):
    dinv = dinv_ref[...]
    h2 = jnp.maximum(dinv * (p_ref[0] + p_ref[1] + g2_ref[...]) + c2b[...], 0.0)
    y_ref[...] = jax.nn.sigmoid(h2 @ ow[...] + ob[...])


def _full(shape):
    return pl.BlockSpec(shape, lambda i: (0,) * len(shape))


def _rows(width):
    return pl.BlockSpec((BN, width), lambda i: (i, 0))


_stage1 = pl.pallas_call(
    _stage1_body,
    grid=(_GRID,),
    in_specs=[
        _rows(NODE_IN),
        _rows(TOPO_IN),
        pl.BlockSpec((NC, BN, 16), lambda i: (0, i, 0)),
        _full((TOPO_IN, 32)), _full((1, 32)),
        _full((32, 32)), _full((1, 32)),
        _full((NODE_IN + 32, HIDDEN)),
    ],
    out_specs=[_rows(HIDDEN), _rows(1)],
    out_shape=[
        jax.ShapeDtypeStruct((N_NODES, HIDDEN), jnp.float32),
        jax.ShapeDtypeStruct((N_NODES, 1), jnp.float32),
    ],
)

_stage2 = pl.pallas_call(
    _stage2_body,
    grid=(_GRID,),
    in_specs=[
        pl.BlockSpec((NC, BN, HIDDEN), lambda i: (0, i, 0)),
        _rows(HIDDEN),
        _rows(1),
        _full((HIDDEN, HIDDEN)), _full((1, HIDDEN)),
    ],
    out_specs=[_rows(HIDDEN)],
    out_shape=[jax.ShapeDtypeStruct((N_NODES, HIDDEN), jnp.float32)],
)

_stage3 = pl.pallas_call(
    _stage3_body,
    grid=(_GRID,),
    in_specs=[
        pl.BlockSpec((NC, BN, HIDDEN), lambda i: (0, i, 0)),
        _rows(HIDDEN),
        _rows(1),
        _full((1, HIDDEN)), _full((HIDDEN, 1)), _full((1, 1)),
    ],
    out_specs=[_rows(1)],
    out_shape=[jax.ShapeDtypeStruct((N_NODES, 1), jnp.float32)],
)


def kernel(x, topo, edge_index, topo_W1, topo_b1, topo_W2, topo_b2,
           conv1_W, conv1_b, conv2_W, conv2_b, out_W, out_b):
    src = edge_index[0].astype(jnp.int32)
    dst = edge_index[1].astype(jnp.int32)
    zeros16 = jnp.zeros((RPT, 16), jnp.float32)
    zeros64 = jnp.zeros((RPT, HIDDEN), jnp.float32)
    ones16 = jnp.ones((CH, 16), jnp.float32)

    degp = _deg_kernel(dst, ones16, zeros16)
    g1, dinv = _stage1(x, topo, degp,
                       topo_W1, topo_b1.reshape(1, -1),
                       topo_W2, topo_b2.reshape(1, -1), conv1_W)
    p1 = _agg_kernel(g1, src, dst, zeros64)
    (g2,) = _stage2(p1, g1, dinv, conv2_W, conv1_b.reshape(1, -1))
    p2 = _agg_kernel(g2, src, dst, zeros64)
    (y,) = _stage3(p2, g2, dinv, conv2_b.reshape(1, -1),
                   out_W, out_b.reshape(1, 1))
    return y


# speedup vs baseline: 28.7296x; 2.1994x over previous
import functools

import jax
import jax.numpy as jnp
from jax import lax
from jax.experimental import pallas as pl
from jax.experimental.pallas import tpu as pltpu
from jax.experimental.pallas import tpu_sc as plsc

N_NODES = 10000
N_EDGES = 320000
NODE_IN = 128
TOPO_IN = 16
HIDDEN = 64

NC = 2
NS = 16
NW = NC * NS
EPW = N_EDGES // NW
CH = 125
NCH = EPW // CH
RPT = 640
NP = RPT * NS

_mesh = plsc.VectorSubcoreMesh(
    core_axis_name="c", subcore_axis_name="s", num_cores=NC, num_subcores=NS)
_sc_params = pltpu.CompilerParams(use_tc_tiling_on_sc=False)


@functools.partial(
    pl.kernel,
    out_type=jax.ShapeDtypeStruct((NC, NP, 16), jnp.float32),
    mesh=_mesh,
    scratch_types=[
        pltpu.VMEM_SHARED((NP, 16), jnp.float32),
        pltpu.VMEM((NCH, CH), jnp.int32),
        pltpu.VMEM((CH, 16), jnp.float32),
    ],
    compiler_params=_sc_params,
)
def _deg_kernel(dst_hbm, ones_hbm, zeros_hbm, out_hbm, tab, didx, ones_v):
    c = lax.axis_index("c")
    s = lax.axis_index("s")
    w = s * NC + c
    pltpu.sync_copy(zeros_hbm, tab.at[pl.ds(s * RPT, RPT)])
    pltpu.sync_copy(ones_hbm, ones_v)
    pltpu.sync_copy(dst_hbm.at[w], didx)
    plsc.subcore_barrier()

    def body(i, carry):
        pltpu.sync_copy(ones_v, tab.at[didx.at[i]], add=True)
        return carry

    lax.fori_loop(0, NCH, body, 0)
    plsc.subcore_barrier()
    pltpu.sync_copy(tab.at[pl.ds(s * RPT, RPT)],
                    out_hbm.at[c, pl.ds(s * RPT, RPT)])


@functools.partial(
    pl.kernel,
    out_type=jax.ShapeDtypeStruct((NC, NP, HIDDEN), jnp.float32),
    mesh=_mesh,
    scratch_types=[
        pltpu.VMEM_SHARED((NP, HIDDEN), jnp.float32),
        pltpu.VMEM((NCH, CH), jnp.int32),
        pltpu.VMEM((NCH, CH), jnp.int32),
        pltpu.VMEM((CH, HIDDEN), jnp.float32),
        pltpu.VMEM((CH, HIDDEN), jnp.float32),
        pltpu.SemaphoreType.DMA,
        pltpu.SemaphoreType.DMA,
    ],
    compiler_params=_sc_params,
)
def _agg_kernel(g_hbm, src_hbm, dst_hbm, zeros_hbm, out_hbm,
                acc, sidx, didx, rows0, rows1, sem0, sem1):
    c = lax.axis_index("c")
    s = lax.axis_index("s")
    w = s * NC + c
    pltpu.sync_copy(zeros_hbm, acc.at[pl.ds(s * RPT, RPT)])
    pltpu.sync_copy(src_hbm.at[w], sidx)
    pltpu.sync_copy(dst_hbm.at[w], didx)
    plsc.subcore_barrier()

    pltpu.async_copy(g_hbm.at[sidx.at[0]], rows0, sem0)

    def body(i2, carry):
        a = 2 * i2
        b = a + 1
        pltpu.make_async_copy(g_hbm.at[sidx.at[a]], rows0, sem0).wait()
        pltpu.async_copy(g_hbm.at[sidx.at[b]], rows1, sem1)
        pltpu.sync_copy(rows0, acc.at[didx.at[a]], add=True)
        pltpu.make_async_copy(g_hbm.at[sidx.at[b]], rows1, sem1).wait()

        @pl.when(i2 < NCH // 2 - 1)
        def _():
            pltpu.async_copy(g_hbm.at[sidx.at[a + 2]], rows0, sem0)

        pltpu.sync_copy(rows1, acc.at[didx.at[b]], add=True)
        return carry

    lax.fori_loop(0, NCH // 2, body, 0)
    plsc.subcore_barrier()
    pltpu.sync_copy(acc.at[pl.ds(s * RPT, RPT)],
                    out_hbm.at[c, pl.ds(s * RPT, RPT)])


BN = 1000
_GRID = N_NODES // BN


def _stage1a_body(x_ref, topo_ref, tw1, tb1, tw2, tb2, c1w, hw_ref):
    z = jnp.maximum(topo_ref[...] @ tw1[...] + tb1[...], 0.0)
    z = jnp.maximum(z @ tw2[...] + tb2[...], 0.0)
    h = jnp.concatenate([x_ref[...], z], axis=1)
    hw_ref[...] = h @ c1w[...]


def _stage1b_body(degp_ref, hw_ref, g1_ref, dinv_ref):
    deg = degp_ref[0, :, 0:1] + degp_ref[1, :, 0:1] + 1.0
    dinv = lax.rsqrt(deg)
    g1_ref[...] = dinv * hw_ref[...]
    dinv_ref[...] = dinv


def _stage2_body(p_ref, g1_ref, dinv_ref, c2w, c1b, g2_ref):
    dinv = dinv_ref[...]
    h1 = jnp.maximum(dinv * (p_ref[0] + p_ref[1] + g1_ref[...]) + c1b[...], 0.0)
    g2_ref[...] = dinv * (h1 @ c2w[...])


def _stage3_body(p_ref, g2_ref, dinv_ref, c2b, ow, ob, y_ref):
    dinv = dinv_ref[...]
    h2 = jnp.maximum(dinv * (p_ref[0] + p_ref[1] + g2_ref[...]) + c2b[...], 0.0)
    y_ref[...] = jax.nn.sigmoid(h2 @ ow[...] + ob[...])


def _full(shape):
    return pl.BlockSpec(shape, lambda i: (0,) * len(shape))


def _rows(width):
    return pl.BlockSpec((BN, width), lambda i: (i, 0))


_stage1a = pl.pallas_call(
    _stage1a_body,
    grid=(_GRID,),
    in_specs=[
        _rows(NODE_IN),
        _rows(TOPO_IN),
        _full((TOPO_IN, 32)), _full((1, 32)),
        _full((32, 32)), _full((1, 32)),
        _full((NODE_IN + 32, HIDDEN)),
    ],
    out_specs=[_rows(HIDDEN)],
    out_shape=[jax.ShapeDtypeStruct((N_NODES, HIDDEN), jnp.float32)],
)

_stage1b = pl.pallas_call(
    _stage1b_body,
    grid=(_GRID,),
    in_specs=[
        pl.BlockSpec((NC, BN, 16), lambda i: (0, i, 0)),
        _rows(HIDDEN),
    ],
    out_specs=[_rows(HIDDEN), _rows(1)],
    out_shape=[
        jax.ShapeDtypeStruct((N_NODES, HIDDEN), jnp.float32),
        jax.ShapeDtypeStruct((N_NODES, 1), jnp.float32),
    ],
)

_stage2 = pl.pallas_call(
    _stage2_body,
    grid=(_GRID,),
    in_specs=[
        pl.BlockSpec((NC, BN, HIDDEN), lambda i: (0, i, 0)),
        _rows(HIDDEN),
        _rows(1),
        _full((HIDDEN, HIDDEN)), _full((1, HIDDEN)),
    ],
    out_specs=[_rows(HIDDEN)],
    out_shape=[jax.ShapeDtypeStruct((N_NODES, HIDDEN), jnp.float32)],
)

_stage3 = pl.pallas_call(
    _stage3_body,
    grid=(_GRID,),
    in_specs=[
        pl.BlockSpec((NC, BN, HIDDEN), lambda i: (0, i, 0)),
        _rows(HIDDEN),
        _rows(1),
        _full((1, HIDDEN)), _full((HIDDEN, 1)), _full((1, 1)),
    ],
    out_specs=[_rows(1)],
    out_shape=[jax.ShapeDtypeStruct((N_NODES, 1), jnp.float32)],
)


def kernel(x, topo, edge_index, topo_W1, topo_b1, topo_W2, topo_b2,
           conv1_W, conv1_b, conv2_W, conv2_b, out_W, out_b):
    src = edge_index[0].astype(jnp.int32).reshape(NW, NCH, CH)
    dst = edge_index[1].astype(jnp.int32).reshape(NW, NCH, CH)
    zeros16 = jnp.zeros((RPT, 16), jnp.float32)
    zeros64 = jnp.zeros((RPT, HIDDEN), jnp.float32)
    ones16 = jnp.ones((CH, 16), jnp.float32)

    degp = _deg_kernel(dst, ones16, zeros16)
    (hw1,) = _stage1a(x, topo,
                      topo_W1, topo_b1.reshape(1, -1),
                      topo_W2, topo_b2.reshape(1, -1), conv1_W)
    g1, dinv = _stage1b(degp, hw1)
    p1 = _agg_kernel(g1, src, dst, zeros64)
    (g2,) = _stage2(p1, g1, dinv, conv2_W, conv1_b.reshape(1, -1))
    p2 = _agg_kernel(g2, src, dst, zeros64)
    (y,) = _stage3(p2, g2, dinv, conv2_b.reshape(1, -1),
                   out_W, out_b.reshape(1, 1))
    return y


# speedup vs baseline: 36.7880x; 1.2805x over previous
import functools

import jax
import jax.numpy as jnp
from jax import lax
from jax.experimental import pallas as pl
from jax.experimental.pallas import tpu as pltpu
from jax.experimental.pallas import tpu_sc as plsc

N_NODES = 10000
N_EDGES = 320000
NODE_IN = 128
TOPO_IN = 16
HIDDEN = 64

NC = 2
NS = 16
NW = NC * NS
EPW = N_EDGES // NW
CH = 125
NCH = EPW // CH
RPT = 640
NP = RPT * NS

_mesh = plsc.VectorSubcoreMesh(
    core_axis_name="c", subcore_axis_name="s", num_cores=NC, num_subcores=NS)
_sc_params = pltpu.CompilerParams(use_tc_tiling_on_sc=False)


@functools.partial(
    pl.kernel,
    out_type=jax.ShapeDtypeStruct((NC, NP, 16), jnp.float32),
    mesh=_mesh,
    scratch_types=[
        pltpu.VMEM_SHARED((NP, 16), jnp.float32),
        pltpu.VMEM((NCH, CH), jnp.int32),
        pltpu.VMEM((CH, 16), jnp.float32),
        pltpu.SemaphoreType.DMA,
    ],
    compiler_params=_sc_params,
)
def _deg_kernel(dst_hbm, ones_hbm, zeros_hbm, out_hbm, tab, didx, ones_v, semd):
    c = lax.axis_index("c")
    s = lax.axis_index("s")
    w = s * NC + c
    pltpu.sync_copy(zeros_hbm, tab.at[pl.ds(s * RPT, RPT)])
    pltpu.sync_copy(ones_hbm, ones_v)
    pltpu.sync_copy(dst_hbm.at[w], didx)
    plsc.subcore_barrier()

    def body(g, carry):
        for j in range(8):
            pltpu.async_copy(ones_v, tab.at[didx.at[g * 8 + j]], semd,
                             add=True)
        for j in range(8):
            pltpu.make_async_copy(ones_v, tab.at[didx.at[g * 8 + j]],
                                  semd).wait()
        return carry

    lax.fori_loop(0, NCH // 8, body, 0)
    plsc.subcore_barrier()
    pltpu.sync_copy(tab.at[pl.ds(s * RPT, RPT)],
                    out_hbm.at[c, pl.ds(s * RPT, RPT)])


@functools.partial(
    pl.kernel,
    out_type=jax.ShapeDtypeStruct((NC, NP, HIDDEN), jnp.float32),
    mesh=_mesh,
    scratch_types=[
        pltpu.VMEM_SHARED((NP, HIDDEN), jnp.float32),
        pltpu.VMEM((NCH, CH), jnp.int32),
        pltpu.VMEM((NCH, CH), jnp.int32),
        pltpu.VMEM((CH, HIDDEN), jnp.float32),
        pltpu.VMEM((CH, HIDDEN), jnp.float32),
        pltpu.VMEM((CH, HIDDEN), jnp.float32),
        pltpu.VMEM((CH, HIDDEN), jnp.float32),
        pltpu.SemaphoreType.DMA,
        pltpu.SemaphoreType.DMA,
        pltpu.SemaphoreType.DMA,
        pltpu.SemaphoreType.DMA,
    ],
    compiler_params=_sc_params,
)
def _agg_kernel(g_hbm, src_hbm, dst_hbm, zeros_hbm, out_hbm,
                acc, sidx, didx, rows0, rows1, rows2, rows3,
                sem0, sem1, sem2, sem3):
    c = lax.axis_index("c")
    s = lax.axis_index("s")
    w = s * NC + c
    pltpu.sync_copy(zeros_hbm, acc.at[pl.ds(s * RPT, RPT)])
    pltpu.sync_copy(src_hbm.at[w], sidx)
    pltpu.sync_copy(dst_hbm.at[w], didx)
    plsc.subcore_barrier()

    bufs = (rows0, rows1, rows2, rows3)
    sems = (sem0, sem1, sem2, sem3)
    pltpu.async_copy(g_hbm.at[sidx.at[0]], rows0, sem0)
    pltpu.async_copy(g_hbm.at[sidx.at[1]], rows1, sem1)

    def body(i4, carry):
        q0 = 4 * i4
        for j in range(4):
            q = q0 + j
            buf, sem = bufs[j], sems[j]
            nbuf, nsem = bufs[(j + 2) % 4], sems[(j + 2) % 4]
            pltpu.make_async_copy(g_hbm.at[sidx.at[q]], buf, sem).wait()

            @pl.when(q + 2 < NCH)
            def _():
                nq = jnp.minimum(q + 2, NCH - 1)
                pltpu.async_copy(g_hbm.at[sidx.at[nq]], nbuf, nsem)

            pltpu.sync_copy(buf, acc.at[didx.at[q]], add=True)
        return carry

    lax.fori_loop(0, NCH // 4, body, 0)
    plsc.subcore_barrier()
    pltpu.sync_copy(acc.at[pl.ds(s * RPT, RPT)],
                    out_hbm.at[c, pl.ds(s * RPT, RPT)])


BN = 1000
_GRID = N_NODES // BN


def _stage1_body(x_ref, topo_ref, degp_ref, tw1, tb1, tw2, tb2, c1w,
                 g1_ref, dinv_ref):
    z = jnp.maximum(topo_ref[...] @ tw1[...] + tb1[...], 0.0)
    z = jnp.maximum(z @ tw2[...] + tb2[...], 0.0)
    deg = degp_ref[0, :, 0:1] + degp_ref[1, :, 0:1] + 1.0
    dinv = lax.rsqrt(deg)
    h = jnp.concatenate([x_ref[...], z], axis=1)
    g1_ref[...] = dinv * (h @ c1w[...])
    dinv_ref[...] = dinv


def _stage2_body(p_ref, g1_ref, dinv_ref, c2w, c1b, g2_ref):
    dinv = dinv_ref[...]
    h1 = jnp.maximum(dinv * (p_ref[0] + p_ref[1] + g1_ref[...]) + c1b[...], 0.0)
    g2_ref[...] = dinv * (h1 @ c2w[...])


def _stage3_body(p_ref, g2_ref, dinv_ref, c2b, ow, ob, y_ref):
    dinv = dinv_ref[...]
    h2 = jnp.maximum(dinv * (p_ref[0] + p_ref[1] + g2_ref[...]) + c2b[...], 0.0)
    y_ref[...] = jax.nn.sigmoid(h2 @ ow[...] + ob[...])


def _full(shape):
    return pl.BlockSpec(shape, lambda i: (0,) * len(shape))


def _rows(width):
    return pl.BlockSpec((BN, width), lambda i: (i, 0))


_stage1 = pl.pallas_call(
    _stage1_body,
    grid=(_GRID,),
    in_specs=[
        _rows(NODE_IN),
        _rows(TOPO_IN),
        pl.BlockSpec((NC, BN, 16), lambda i: (0, i, 0)),
        _full((TOPO_IN, 32)), _full((1, 32)),
        _full((32, 32)), _full((1, 32)),
        _full((NODE_IN + 32, HIDDEN)),
    ],
    out_specs=[_rows(HIDDEN), _rows(1)],
    out_shape=[
        jax.ShapeDtypeStruct((N_NODES, HIDDEN), jnp.float32),
        jax.ShapeDtypeStruct((N_NODES, 1), jnp.float32),
    ],
)

_stage2 = pl.pallas_call(
    _stage2_body,
    grid=(_GRID,),
    in_specs=[
        pl.BlockSpec((NC, BN, HIDDEN), lambda i: (0, i, 0)),
        _rows(HIDDEN),
        _rows(1),
        _full((HIDDEN, HIDDEN)), _full((1, HIDDEN)),
    ],
    out_specs=[_rows(HIDDEN)],
    out_shape=[jax.ShapeDtypeStruct((N_NODES, HIDDEN), jnp.float32)],
)

_stage3 = pl.pallas_call(
    _stage3_body,
    grid=(_GRID,),
    in_specs=[
        pl.BlockSpec((NC, BN, HIDDEN), lambda i: (0, i, 0)),
        _rows(HIDDEN),
        _rows(1),
        _full((1, HIDDEN)), _full((HIDDEN, 1)), _full((1, 1)),
    ],
    out_specs=[_rows(1)],
    out_shape=[jax.ShapeDtypeStruct((N_NODES, 1), jnp.float32)],
)


def kernel(x, topo, edge_index, topo_W1, topo_b1, topo_W2, topo_b2,
           conv1_W, conv1_b, conv2_W, conv2_b, out_W, out_b):
    src = edge_index[0].astype(jnp.int32).reshape(NW, NCH, CH)
    dst = edge_index[1].astype(jnp.int32).reshape(NW, NCH, CH)
    zeros16 = jnp.zeros((RPT, 16), jnp.float32)
    zeros64 = jnp.zeros((RPT, HIDDEN), jnp.float32)
    ones16 = jnp.ones((CH, 16), jnp.float32)

    degp = _deg_kernel(dst, ones16, zeros16)
    g1, dinv = _stage1(x, topo, degp,
                       topo_W1, topo_b1.reshape(1, -1),
                       topo_W2, topo_b2.reshape(1, -1), conv1_W)
    p1 = _agg_kernel(g1, src, dst, zeros64)
    (g2,) = _stage2(p1, g1, dinv, conv2_W, conv1_b.reshape(1, -1))
    p2 = _agg_kernel(g2, src, dst, zeros64)
    (y,) = _stage3(p2, g2, dinv, conv2_b.reshape(1, -1),
                   out_W, out_b.reshape(1, 1))
    return y


# speedup vs baseline: 39.5329x; 1.0746x over previous
import functools

import jax
import jax.numpy as jnp
from jax import lax
from jax.experimental import pallas as pl
from jax.experimental.pallas import tpu as pltpu
from jax.experimental.pallas import tpu_sc as plsc

N_NODES = 10000
N_EDGES = 320000
NODE_IN = 128
TOPO_IN = 16
HIDDEN = 64

NC = 2
NS = 16
NW = NC * NS
EPW = N_EDGES // NW
CH = 125
NCH = EPW // CH
RPT = 640
NP = RPT * NS

_mesh = plsc.VectorSubcoreMesh(
    core_axis_name="c", subcore_axis_name="s", num_cores=NC, num_subcores=NS)
_sc_params = pltpu.CompilerParams(use_tc_tiling_on_sc=False)


@functools.partial(
    pl.kernel,
    out_type=jax.ShapeDtypeStruct((NC, NP, 16), jnp.float32),
    mesh=_mesh,
    scratch_types=[
        pltpu.VMEM_SHARED((NP, 16), jnp.float32),
        pltpu.VMEM((NCH, CH), jnp.int32),
        pltpu.VMEM((CH, 16), jnp.float32),
        pltpu.SemaphoreType.DMA,
    ],
    compiler_params=_sc_params,
)
def _deg_kernel(ei_hbm, ones_hbm, zeros_hbm, out_hbm, tab, didx, ones_v, semd):
    c = lax.axis_index("c")
    s = lax.axis_index("s")
    w = s * NC + c
    pltpu.sync_copy(zeros_hbm, tab.at[pl.ds(s * RPT, RPT)])
    pltpu.sync_copy(ones_hbm, ones_v)
    pltpu.sync_copy(ei_hbm.at[1, w], didx)
    plsc.subcore_barrier()

    def body(g, carry):
        for j in range(8):
            pltpu.async_copy(ones_v, tab.at[didx.at[g * 8 + j]], semd,
                             add=True)
        for j in range(8):
            pltpu.make_async_copy(ones_v, tab.at[didx.at[g * 8 + j]],
                                  semd).wait()
        return carry

    lax.fori_loop(0, NCH // 8, body, 0)
    plsc.subcore_barrier()
    pltpu.sync_copy(tab.at[pl.ds(s * RPT, RPT)],
                    out_hbm.at[c, pl.ds(s * RPT, RPT)])


@functools.partial(
    pl.kernel,
    out_type=jax.ShapeDtypeStruct((NC, NP, HIDDEN), jnp.float32),
    mesh=_mesh,
    scratch_types=[
        pltpu.VMEM_SHARED((NP, HIDDEN), jnp.float32),
        pltpu.VMEM((NCH, CH), jnp.int32),
        pltpu.VMEM((NCH, CH), jnp.int32),
        pltpu.VMEM((CH, HIDDEN), jnp.float32),
        pltpu.VMEM((CH, HIDDEN), jnp.float32),
        pltpu.VMEM((CH, HIDDEN), jnp.float32),
        pltpu.VMEM((CH, HIDDEN), jnp.float32),
        pltpu.SemaphoreType.DMA,
        pltpu.SemaphoreType.DMA,
        pltpu.SemaphoreType.DMA,
        pltpu.SemaphoreType.DMA,
    ],
    compiler_params=_sc_params,
)
def _agg_kernel(g_hbm, ei_hbm, zeros_hbm, out_hbm,
                acc, sidx, didx, rows0, rows1, rows2, rows3,
                sem0, sem1, sem2, sem3):
    c = lax.axis_index("c")
    s = lax.axis_index("s")
    w = s * NC + c
    pltpu.sync_copy(zeros_hbm, acc.at[pl.ds(s * RPT, RPT)])
    pltpu.sync_copy(ei_hbm.at[0, w], sidx)
    pltpu.sync_copy(ei_hbm.at[1, w], didx)
    plsc.subcore_barrier()

    bufs = (rows0, rows1, rows2, rows3)
    sems = (sem0, sem1, sem2, sem3)
    pltpu.async_copy(g_hbm.at[sidx.at[0]], rows0, sem0)
    pltpu.async_copy(g_hbm.at[sidx.at[1]], rows1, sem1)
    pltpu.async_copy(g_hbm.at[sidx.at[2]], rows2, sem2)

    def body(i4, carry):
        q0 = 4 * i4
        for j in range(4):
            q = q0 + j
            buf, sem = bufs[j], sems[j]
            nbuf, nsem = bufs[(j + 3) % 4], sems[(j + 3) % 4]
            pltpu.make_async_copy(g_hbm.at[sidx.at[q]], buf, sem).wait()

            @pl.when(q + 3 < NCH)
            def _():
                nq = jnp.minimum(q + 3, NCH - 1)
                pltpu.async_copy(g_hbm.at[sidx.at[nq]], nbuf, nsem)

            pltpu.sync_copy(buf, acc.at[didx.at[q]], add=True)
        return carry

    lax.fori_loop(0, NCH // 4, body, 0)
    plsc.subcore_barrier()
    pltpu.sync_copy(acc.at[pl.ds(s * RPT, RPT)],
                    out_hbm.at[c, pl.ds(s * RPT, RPT)])


BN = 1000
_GRID = N_NODES // BN


def _stage1_body(x_ref, topo_ref, degp_ref, tw1, tb1, tw2, tb2, c1w,
                 g1_ref, dinv_ref):
    z = jnp.maximum(topo_ref[...] @ tw1[...] + tb1[...], 0.0)
    z = jnp.maximum(z @ tw2[...] + tb2[...], 0.0)
    deg = degp_ref[0, :, 0:1] + degp_ref[1, :, 0:1] + 1.0
    dinv = lax.rsqrt(deg)
    h = jnp.concatenate([x_ref[...], z], axis=1)
    g1_ref[...] = dinv * (h @ c1w[...])
    dinv_ref[...] = dinv


def _stage2_body(p_ref, g1_ref, dinv_ref, c2w, c1b, g2_ref):
    dinv = dinv_ref[...]
    h1 = jnp.maximum(dinv * (p_ref[0] + p_ref[1] + g1_ref[...]) + c1b[...], 0.0)
    g2_ref[...] = dinv * (h1 @ c2w[...])


def _stage3_body(p_ref, g2_ref, dinv_ref, c2b, ow, ob, y_ref):
    dinv = dinv_ref[...]
    h2 = jnp.maximum(dinv * (p_ref[0] + p_ref[1] + g2_ref[...]) + c2b[...], 0.0)
    y_ref[...] = jax.nn.sigmoid(h2 @ ow[...] + ob[...])


def _full(shape):
    return pl.BlockSpec(shape, lambda i: (0,) * len(shape))


def _rows(width):
    return pl.BlockSpec((BN, width), lambda i: (i, 0))


_stage1 = pl.pallas_call(
    _stage1_body,
    grid=(_GRID,),
    in_specs=[
        _rows(NODE_IN),
        _rows(TOPO_IN),
        pl.BlockSpec((NC, BN, 16), lambda i: (0, i, 0)),
        _full((TOPO_IN, 32)), _full((1, 32)),
        _full((32, 32)), _full((1, 32)),
        _full((NODE_IN + 32, HIDDEN)),
    ],
    out_specs=[_rows(HIDDEN), _rows(1)],
    out_shape=[
        jax.ShapeDtypeStruct((N_NODES, HIDDEN), jnp.float32),
        jax.ShapeDtypeStruct((N_NODES, 1), jnp.float32),
    ],
)

_stage2 = pl.pallas_call(
    _stage2_body,
    grid=(_GRID,),
    in_specs=[
        pl.BlockSpec((NC, BN, HIDDEN), lambda i: (0, i, 0)),
        _rows(HIDDEN),
        _rows(1),
        _full((HIDDEN, HIDDEN)), _full((1, HIDDEN)),
    ],
    out_specs=[_rows(HIDDEN)],
    out_shape=[jax.ShapeDtypeStruct((N_NODES, HIDDEN), jnp.float32)],
)

_stage3 = pl.pallas_call(
    _stage3_body,
    grid=(_GRID,),
    in_specs=[
        pl.BlockSpec((NC, BN, HIDDEN), lambda i: (0, i, 0)),
        _rows(HIDDEN),
        _rows(1),
        _full((1, HIDDEN)), _full((HIDDEN, 1)), _full((1, 1)),
    ],
    out_specs=[_rows(1)],
    out_shape=[jax.ShapeDtypeStruct((N_NODES, 1), jnp.float32)],
)


def kernel(x, topo, edge_index, topo_W1, topo_b1, topo_W2, topo_b2,
           conv1_W, conv1_b, conv2_W, conv2_b, out_W, out_b):
    ei = edge_index.astype(jnp.int32).reshape(2, NW, NCH, CH)
    zeros16 = jnp.zeros((RPT, 16), jnp.float32)
    zeros64 = jnp.zeros((RPT, HIDDEN), jnp.float32)
    ones16 = jnp.ones((CH, 16), jnp.float32)

    degp = _deg_kernel(ei, ones16, zeros16)
    g1, dinv = _stage1(x, topo, degp,
                       topo_W1, topo_b1.reshape(1, -1),
                       topo_W2, topo_b2.reshape(1, -1), conv1_W)
    p1 = _agg_kernel(g1, ei, zeros64)
    (g2,) = _stage2(p1, g1, dinv, conv2_W, conv1_b.reshape(1, -1))
    p2 = _agg_kernel(g2, ei, zeros64)
    (y,) = _stage3(p2, g2, dinv, conv2_b.reshape(1, -1),
                   out_W, out_b.reshape(1, 1))
    return y


# speedup vs baseline: 41.2467x; 1.0433x over previous
import functools

import jax
import jax.numpy as jnp
from jax import lax
from jax.experimental import pallas as pl
from jax.experimental.pallas import tpu as pltpu
from jax.experimental.pallas import tpu_sc as plsc

N_NODES = 10000
N_EDGES = 320000
NODE_IN = 128
TOPO_IN = 16
HIDDEN = 64

NC = 2
NS = 16
NW = NC * NS
EPW = N_EDGES // NW
CH = 125
NCH = EPW // CH
RPT = 640
NP = RPT * NS

_mesh = plsc.VectorSubcoreMesh(
    core_axis_name="c", subcore_axis_name="s", num_cores=NC, num_subcores=NS)
_sc_params = pltpu.CompilerParams(use_tc_tiling_on_sc=False)


@functools.partial(
    pl.kernel,
    out_type=jax.ShapeDtypeStruct((NC, NP, 16), jnp.float32),
    mesh=_mesh,
    scratch_types=[
        pltpu.VMEM_SHARED((NP, 16), jnp.float32),
        pltpu.VMEM((NCH, CH), jnp.int32),
        pltpu.VMEM((CH, 16), jnp.float32),
        pltpu.SemaphoreType.DMA,
    ],
    compiler_params=_sc_params,
)
def _deg_kernel(ei_hbm, ones_hbm, zeros_hbm, out_hbm, tab, didx, ones_v, semd):
    c = lax.axis_index("c")
    s = lax.axis_index("s")
    w = s * NC + c
    pltpu.sync_copy(zeros_hbm, tab.at[pl.ds(s * RPT, RPT)])
    pltpu.sync_copy(ones_hbm, ones_v)
    pltpu.sync_copy(ei_hbm.at[1, w], didx)
    plsc.subcore_barrier()

    def body(g, carry):
        for j in range(16):
            pltpu.async_copy(ones_v, tab.at[didx.at[g * 16 + j]], semd,
                             add=True)
        for j in range(16):
            pltpu.make_async_copy(ones_v, tab.at[didx.at[g * 16 + j]],
                                  semd).wait()
        return carry

    lax.fori_loop(0, NCH // 16, body, 0)
    plsc.subcore_barrier()
    pltpu.sync_copy(tab.at[pl.ds(s * RPT, RPT)],
                    out_hbm.at[c, pl.ds(s * RPT, RPT)])


@functools.partial(
    pl.kernel,
    out_type=jax.ShapeDtypeStruct((NC, NP, HIDDEN), jnp.float32),
    mesh=_mesh,
    scratch_types=[
        pltpu.VMEM_SHARED((NP, HIDDEN), jnp.float32),
        pltpu.VMEM((NCH, CH), jnp.int32),
        pltpu.VMEM((NCH, CH), jnp.int32),
        pltpu.VMEM((CH, HIDDEN), jnp.float32),
        pltpu.VMEM((CH, HIDDEN), jnp.float32),
        pltpu.VMEM((CH, HIDDEN), jnp.float32),
        pltpu.VMEM((CH, HIDDEN), jnp.float32),
        pltpu.SemaphoreType.DMA,
        pltpu.SemaphoreType.DMA,
        pltpu.SemaphoreType.DMA,
        pltpu.SemaphoreType.DMA,
    ],
    compiler_params=_sc_params,
)
def _agg_kernel(g_hbm, ei_hbm, zeros_hbm, out_hbm,
                acc, sidx, didx, rows0, rows1, rows2, rows3,
                sem0, sem1, sem2, sem3):
    c = lax.axis_index("c")
    s = lax.axis_index("s")
    w = s * NC + c
    pltpu.sync_copy(zeros_hbm, acc.at[pl.ds(s * RPT, RPT)])
    pltpu.sync_copy(ei_hbm.at[0, w], sidx)
    pltpu.sync_copy(ei_hbm.at[1, w], didx)
    plsc.subcore_barrier()

    bufs = (rows0, rows1, rows2, rows3)
    sems = (sem0, sem1, sem2, sem3)
    pltpu.async_copy(g_hbm.at[sidx.at[0]], rows0, sem0)
    pltpu.async_copy(g_hbm.at[sidx.at[1]], rows1, sem1)
    pltpu.async_copy(g_hbm.at[sidx.at[2]], rows2, sem2)

    def body(i4, carry):
        q0 = 4 * i4
        for j in range(4):
            q = q0 + j
            buf, sem = bufs[j], sems[j]
            nbuf, nsem = bufs[(j + 3) % 4], sems[(j + 3) % 4]
            pltpu.make_async_copy(g_hbm.at[sidx.at[q]], buf, sem).wait()

            @pl.when(q + 3 < NCH)
            def _():
                nq = jnp.minimum(q + 3, NCH - 1)
                pltpu.async_copy(g_hbm.at[sidx.at[nq]], nbuf, nsem)

            pltpu.sync_copy(buf, acc.at[didx.at[q]], add=True)
        return carry

    lax.fori_loop(0, NCH // 4, body, 0)
    plsc.subcore_barrier()
    pltpu.sync_copy(acc.at[pl.ds(s * RPT, RPT)],
                    out_hbm.at[c, pl.ds(s * RPT, RPT)])


BN = 2000
_GRID = N_NODES // BN


def _stage1a_body(x_ref, topo_ref, tw1, tb1, tw2, tb2, c1w, hw_ref):
    z = jnp.maximum(topo_ref[...] @ tw1[...] + tb1[...], 0.0)
    z = jnp.maximum(z @ tw2[...] + tb2[...], 0.0)
    h = jnp.concatenate([x_ref[...], z], axis=1)
    hw_ref[...] = h @ c1w[...]


def _stage1b_body(degp_ref, hw_ref, g1_ref, dinv_ref):
    deg = degp_ref[0, :, 0:1] + degp_ref[1, :, 0:1] + 1.0
    dinv = lax.rsqrt(deg)
    g1_ref[...] = dinv * hw_ref[...]
    dinv_ref[...] = dinv


def _stage2_body(p_ref, g1_ref, dinv_ref, c2w, c1b, g2_ref):
    dinv = dinv_ref[...]
    h1 = jnp.maximum(dinv * (p_ref[0] + p_ref[1] + g1_ref[...]) + c1b[...], 0.0)
    g2_ref[...] = dinv * (h1 @ c2w[...])


def _stage3_body(p_ref, g2_ref, dinv_ref, c2b, ow, ob, y_ref):
    dinv = dinv_ref[...]
    h2 = jnp.maximum(dinv * (p_ref[0] + p_ref[1] + g2_ref[...]) + c2b[...], 0.0)
    y_ref[...] = jax.nn.sigmoid(h2 @ ow[...] + ob[...])


def _full(shape):
    return pl.BlockSpec(shape, lambda i: (0,) * len(shape))


def _rows(width):
    return pl.BlockSpec((BN, width), lambda i: (i, 0))


_stage1a = pl.pallas_call(
    _stage1a_body,
    grid=(_GRID,),
    in_specs=[
        _rows(NODE_IN),
        _rows(TOPO_IN),
        _full((TOPO_IN, 32)), _full((1, 32)),
        _full((32, 32)), _full((1, 32)),
        _full((NODE_IN + 32, HIDDEN)),
    ],
    out_specs=[_rows(HIDDEN)],
    out_shape=[jax.ShapeDtypeStruct((N_NODES, HIDDEN), jnp.float32)],
)

_stage1b = pl.pallas_call(
    _stage1b_body,
    grid=(_GRID,),
    in_specs=[
        pl.BlockSpec((NC, BN, 16), lambda i: (0, i, 0)),
        _rows(HIDDEN),
    ],
    out_specs=[_rows(HIDDEN), _rows(1)],
    out_shape=[
        jax.ShapeDtypeStruct((N_NODES, HIDDEN), jnp.float32),
        jax.ShapeDtypeStruct((N_NODES, 1), jnp.float32),
    ],
)

_stage2 = pl.pallas_call(
    _stage2_body,
    grid=(_GRID,),
    in_specs=[
        pl.BlockSpec((NC, BN, HIDDEN), lambda i: (0, i, 0)),
        _rows(HIDDEN),
        _rows(1),
        _full((HIDDEN, HIDDEN)), _full((1, HIDDEN)),
    ],
    out_specs=[_rows(HIDDEN)],
    out_shape=[jax.ShapeDtypeStruct((N_NODES, HIDDEN), jnp.float32)],
)

_stage3 = pl.pallas_call(
    _stage3_body,
    grid=(_GRID,),
    in_specs=[
        pl.BlockSpec((NC, BN, HIDDEN), lambda i: (0, i, 0)),
        _rows(HIDDEN),
        _rows(1),
        _full((1, HIDDEN)), _full((HIDDEN, 1)), _full((1, 1)),
    ],
    out_specs=[_rows(1)],
    out_shape=[jax.ShapeDtypeStruct((N_NODES, 1), jnp.float32)],
)


def kernel(x, topo, edge_index, topo_W1, topo_b1, topo_W2, topo_b2,
           conv1_W, conv1_b, conv2_W, conv2_b, out_W, out_b):
    ei = edge_index.astype(jnp.int32).reshape(2, NW, NCH, CH)
    zeros16 = jnp.zeros((RPT, 16), jnp.float32)
    zeros64 = jnp.zeros((RPT, HIDDEN), jnp.float32)
    ones16 = jnp.ones((CH, 16), jnp.float32)

    degp = _deg_kernel(ei, ones16, zeros16)
    (hw1,) = _stage1a(x, topo,
                      topo_W1, topo_b1.reshape(1, -1),
                      topo_W2, topo_b2.reshape(1, -1), conv1_W)
    g1, dinv = _stage1b(degp, hw1)
    p1 = _agg_kernel(g1, ei, zeros64)
    (g2,) = _stage2(p1, g1, dinv, conv2_W, conv1_b.reshape(1, -1))
    p2 = _agg_kernel(g2, ei, zeros64)
    (y,) = _stage3(p2, g2, dinv, conv2_b.reshape(1, -1),
                   out_W, out_b.reshape(1, 1))
    return y


# speedup vs baseline: 42.3811x; 1.0275x over previous
import functools

import jax
import jax.numpy as jnp
from jax import lax
from jax.experimental import pallas as pl
from jax.experimental.pallas import tpu as pltpu
from jax.experimental.pallas import tpu_sc as plsc

N_NODES = 10000
N_EDGES = 320000
NODE_IN = 128
TOPO_IN = 16
HIDDEN = 64

NC = 2
NS = 16
NW = NC * NS
EPW = N_EDGES // NW
CH = 125
NCH = EPW // CH
RPT = 640
NP = RPT * NS

_mesh = plsc.VectorSubcoreMesh(
    core_axis_name="c", subcore_axis_name="s", num_cores=NC, num_subcores=NS)
_sc_params = pltpu.CompilerParams(use_tc_tiling_on_sc=False)


@functools.partial(
    pl.kernel,
    out_type=jax.ShapeDtypeStruct((NC, NP, 16), jnp.float32),
    mesh=_mesh,
    scratch_types=[
        pltpu.VMEM_SHARED((NP, 16), jnp.float32),
        pltpu.VMEM((NCH, CH), jnp.int32),
        pltpu.VMEM((CH, 16), jnp.float32),
        pltpu.SemaphoreType.DMA,
        pltpu.SemaphoreType.DMA,
    ],
    compiler_params=_sc_params,
)
def _deg_kernel(ei_hbm, ones_hbm, zeros_hbm, out_hbm, tab, didx, ones_v,
                semd, semp):
    c = lax.axis_index("c")
    s = lax.axis_index("s")
    w = s * NC + c
    pltpu.async_copy(zeros_hbm, tab.at[pl.ds(s * RPT, RPT)], semp)
    pltpu.async_copy(ones_hbm, ones_v, semp)
    pltpu.async_copy(ei_hbm.at[1, w], didx, semp)
    pltpu.make_async_copy(zeros_hbm, tab.at[pl.ds(s * RPT, RPT)], semp).wait()
    pltpu.make_async_copy(ones_hbm, ones_v, semp).wait()
    pltpu.make_async_copy(ei_hbm.at[1, w], didx, semp).wait()
    plsc.subcore_barrier()

    def body(g, carry):
        for j in range(16):
            pltpu.async_copy(ones_v, tab.at[didx.at[g * 16 + j]], semd,
                             add=True)
        for j in range(16):
            pltpu.make_async_copy(ones_v, tab.at[didx.at[g * 16 + j]],
                                  semd).wait()
        return carry

    lax.fori_loop(0, NCH // 16, body, 0)
    plsc.subcore_barrier()
    pltpu.sync_copy(tab.at[pl.ds(s * RPT, RPT)],
                    out_hbm.at[c, pl.ds(s * RPT, RPT)])


@functools.partial(
    pl.kernel,
    out_type=jax.ShapeDtypeStruct((NC, NP, HIDDEN), jnp.float32),
    mesh=_mesh,
    scratch_types=[
        pltpu.VMEM_SHARED((NP, HIDDEN), jnp.float32),
        pltpu.VMEM((NCH, CH), jnp.int32),
        pltpu.VMEM((NCH, CH), jnp.int32),
        pltpu.VMEM((CH, HIDDEN), jnp.float32),
        pltpu.VMEM((CH, HIDDEN), jnp.float32),
        pltpu.VMEM((CH, HIDDEN), jnp.float32),
        pltpu.VMEM((CH, HIDDEN), jnp.float32),
        pltpu.SemaphoreType.DMA,
        pltpu.SemaphoreType.DMA,
        pltpu.SemaphoreType.DMA,
        pltpu.SemaphoreType.DMA,
    ],
    compiler_params=_sc_params,
)
def _agg_kernel(g_hbm, ei_hbm, zeros_hbm, out_hbm,
                acc, sidx, didx, rows0, rows1, rows2, rows3,
                sem0, sem1, sem2, sem3):
    c = lax.axis_index("c")
    s = lax.axis_index("s")
    w = s * NC + c
    pltpu.async_copy(zeros_hbm, acc.at[pl.ds(s * RPT, RPT)], sem0)
    pltpu.async_copy(ei_hbm.at[0, w], sidx, sem1)
    pltpu.async_copy(ei_hbm.at[1, w], didx, sem2)
    pltpu.make_async_copy(zeros_hbm, acc.at[pl.ds(s * RPT, RPT)], sem0).wait()
    pltpu.make_async_copy(ei_hbm.at[0, w], sidx, sem1).wait()
    pltpu.make_async_copy(ei_hbm.at[1, w], didx, sem2).wait()
    plsc.subcore_barrier()

    bufs = (rows0, rows1, rows2, rows3)
    sems = (sem0, sem1, sem2, sem3)
    pltpu.async_copy(g_hbm.at[sidx.at[0]], rows0, sem0)
    pltpu.async_copy(g_hbm.at[sidx.at[1]], rows1, sem1)
    pltpu.async_copy(g_hbm.at[sidx.at[2]], rows2, sem2)

    def body(i4, carry):
        q0 = 4 * i4
        for j in range(4):
            q = q0 + j
            buf, sem = bufs[j], sems[j]
            nbuf, nsem = bufs[(j + 3) % 4], sems[(j + 3) % 4]
            pltpu.make_async_copy(g_hbm.at[sidx.at[q]], buf, sem).wait()

            @pl.when(q + 3 < NCH)
            def _():
                nq = jnp.minimum(q + 3, NCH - 1)
                pltpu.async_copy(g_hbm.at[sidx.at[nq]], nbuf, nsem)

            pltpu.sync_copy(buf, acc.at[didx.at[q]], add=True)
        return carry

    lax.fori_loop(0, NCH // 4, body, 0)
    plsc.subcore_barrier()
    pltpu.sync_copy(acc.at[pl.ds(s * RPT, RPT)],
                    out_hbm.at[c, pl.ds(s * RPT, RPT)])


BN = 2000
_GRID = N_NODES // BN


def _stage1a_body(x_ref, topo_ref, tw1, tb1, tw2, tb2, c1w, hw_ref):
    z = jnp.maximum(topo_ref[...] @ tw1[...] + tb1[...], 0.0)
    z = jnp.maximum(z @ tw2[...] + tb2[...], 0.0)
    h = jnp.concatenate([x_ref[...], z], axis=1)
    hw_ref[...] = h @ c1w[...]


def _stage1b_body(degp_ref, hw_ref, g1_ref, dinv_ref):
    deg = degp_ref[0, :, 0:1] + degp_ref[1, :, 0:1] + 1.0
    dinv = lax.rsqrt(deg)
    g1_ref[...] = dinv * hw_ref[...]
    dinv_ref[...] = dinv


def _stage2_body(p_ref, g1_ref, dinv_ref, c2w, c1b, g2_ref):
    dinv = dinv_ref[...]
    h1 = jnp.maximum(dinv * (p_ref[0] + p_ref[1] + g1_ref[...]) + c1b[...], 0.0)
    g2_ref[...] = dinv * (h1 @ c2w[...])


def _stage3_body(p_ref, g2_ref, dinv_ref, c2b, ow, ob, y_ref):
    dinv = dinv_ref[...]
    h2 = jnp.maximum(dinv * (p_ref[0] + p_ref[1] + g2_ref[...]) + c2b[...], 0.0)
    y_ref[...] = jax.nn.sigmoid(h2 @ ow[...] + ob[...])


def _full(shape):
    return pl.BlockSpec(shape, lambda i: (0,) * len(shape))


def _rows(width):
    return pl.BlockSpec((BN, width), lambda i: (i, 0))


_stage1a = pl.pallas_call(
    _stage1a_body,
    grid=(_GRID,),
    in_specs=[
        _rows(NODE_IN),
        _rows(TOPO_IN),
        _full((TOPO_IN, 32)), _full((1, 32)),
        _full((32, 32)), _full((1, 32)),
        _full((NODE_IN + 32, HIDDEN)),
    ],
    out_specs=[_rows(HIDDEN)],
    out_shape=[jax.ShapeDtypeStruct((N_NODES, HIDDEN), jnp.float32)],
)

_stage1b = pl.pallas_call(
    _stage1b_body,
    grid=(_GRID,),
    in_specs=[
        pl.BlockSpec((NC, BN, 16), lambda i: (0, i, 0)),
        _rows(HIDDEN),
    ],
    out_specs=[_rows(HIDDEN), _rows(1)],
    out_shape=[
        jax.ShapeDtypeStruct((N_NODES, HIDDEN), jnp.float32),
        jax.ShapeDtypeStruct((N_NODES, 1), jnp.float32),
    ],
)

_stage2 = pl.pallas_call(
    _stage2_body,
    grid=(_GRID,),
    in_specs=[
        pl.BlockSpec((NC, BN, HIDDEN), lambda i: (0, i, 0)),
        _rows(HIDDEN),
        _rows(1),
        _full((HIDDEN, HIDDEN)), _full((1, HIDDEN)),
    ],
    out_specs=[_rows(HIDDEN)],
    out_shape=[jax.ShapeDtypeStruct((N_NODES, HIDDEN), jnp.float32)],
)

_stage3 = pl.pallas_call(
    _stage3_body,
    grid=(_GRID,),
    in_specs=[
        pl.BlockSpec((NC, BN, HIDDEN), lambda i: (0, i, 0)),
        _rows(HIDDEN),
        _rows(1),
        _full((1, HIDDEN)), _full((HIDDEN, 1)), _full((1, 1)),
    ],
    out_specs=[_rows(1)],
    out_shape=[jax.ShapeDtypeStruct((N_NODES, 1), jnp.float32)],
)


def kernel(x, topo, edge_index, topo_W1, topo_b1, topo_W2, topo_b2,
           conv1_W, conv1_b, conv2_W, conv2_b, out_W, out_b):
    ei = edge_index.astype(jnp.int32).reshape(2, NW, NCH, CH)
    zeros16 = jnp.zeros((RPT, 16), jnp.float32)
    zeros64 = jnp.zeros((RPT, HIDDEN), jnp.float32)
    ones16 = jnp.ones((CH, 16), jnp.float32)

    degp = _deg_kernel(ei, ones16, zeros16)
    (hw1,) = _stage1a(x, topo,
                      topo_W1, topo_b1.reshape(1, -1),
                      topo_W2, topo_b2.reshape(1, -1), conv1_W)
    g1, dinv = _stage1b(degp, hw1)
    p1 = _agg_kernel(g1, ei, zeros64)
    (g2,) = _stage2(p1, g1, dinv, conv2_W, conv1_b.reshape(1, -1))
    p2 = _agg_kernel(g2, ei, zeros64)
    (y,) = _stage3(p2, g2, dinv, conv2_b.reshape(1, -1),
                   out_W, out_b.reshape(1, 1))
    return y
